# baseline (speedup 1.0000x reference)
import functools

import jax
import jax.numpy as jnp
from jax import lax
from jax.experimental import pallas as pl
from jax.experimental.pallas import tpu as pltpu
from jax.experimental.pallas import tpu_sc as plsc


def _gn(x, g, b, eps=1e-5):
    mu = jnp.mean(x, axis=-1, keepdims=True)
    var = jnp.mean((x - mu) ** 2, axis=-1, keepdims=True)
    return (x - mu) / jnp.sqrt(var + eps) * g + b


def _pre_body(agts_ref, ctx_ref, qw_ref, qg_ref, qb_ref, w1q_ref, w1c_ref,
              aw_ref, qc_ref, cc_ref, a0_ref):
    x = agts_ref[...]
    q = jax.nn.relu(_gn(jnp.dot(x, qw_ref[...],
                                preferred_element_type=jnp.float32),
                        qg_ref[...], qb_ref[...]))
    qc_ref[...] = jnp.dot(q, w1q_ref[...], preferred_element_type=jnp.float32)
    cc_ref[...] = jnp.dot(ctx_ref[...], w1c_ref[...],
                          preferred_element_type=jnp.float32)
    a0_ref[...] = jnp.dot(x, aw_ref[...], preferred_element_type=jnp.float32)


def _node_pre(agts, ctx, query_w, query_g, query_b, w1q, w1c, agt_w, bn):
    n, d = agts.shape
    grid = n // bn
    row = pl.BlockSpec((bn, d), lambda i: (i, 0))
    mat = pl.BlockSpec((d, d), lambda i: (0, 0))
    vec = pl.BlockSpec((1, d), lambda i: (0, 0))
    out = jax.ShapeDtypeStruct((n, d), jnp.float32)
    outh = jax.ShapeDtypeStruct((n, d), jnp.bfloat16)
    return pl.pallas_call(
        _pre_body,
        grid=(grid,),
        in_specs=[row, row, mat, vec, vec, mat, mat, mat],
        out_specs=[row, row, row],
        out_shape=[out, out, out],
    )(agts, ctx, query_w, query_g.reshape(1, d), query_b.reshape(1, d),
      w1q, w1c, agt_w)


def _make_gather(n, e, d, nc, ns, ch, be):
    ew = e // (nc * ns)
    nchunks = ew // ch
    assert nchunks % 2 == 1, "pipelined schedule assumes an odd chunk count"
    npairs = (nchunks - 1) // 2
    nblk = e // be
    rpb = be // 8
    rchunk = 80
    mesh = plsc.VectorSubcoreMesh(core_axis_name="c", subcore_axis_name="s")

    @functools.partial(
        pl.kernel,
        mesh=mesh,
        out_type=[
            jax.ShapeDtypeStruct((e, d), jnp.float32),
            jax.ShapeDtypeStruct((e, d), jnp.float32),
            jax.ShapeDtypeStruct((e // 8, 128), jnp.float32),
        ],
        scratch_types=[
            pltpu.VMEM((ch,), jnp.int32),
            pltpu.VMEM((ch,), jnp.int32),
            pltpu.VMEM((ch,), jnp.int32),
            pltpu.VMEM((ch,), jnp.int32),
            pltpu.VMEM((ch, d), jnp.float32),
            pltpu.VMEM((ch, d), jnp.float32),
            pltpu.VMEM((ch, d), jnp.float32),
            pltpu.VMEM((ch, d), jnp.float32),
            pltpu.VMEM((rchunk, 128), jnp.float32),
            pltpu.VMEM((be,), jnp.int32),
            pltpu.VMEM((be,), jnp.int32),
            pltpu.VMEM((2 * n,), jnp.float32),
            pltpu.VMEM((2 * n,), jnp.float32),
            pltpu.SemaphoreType.DMA,
            pltpu.SemaphoreType.DMA,
            pltpu.SemaphoreType.DMA,
            pltpu.SemaphoreType.DMA,
        ],
        compiler_params=pltpu.CompilerParams(needs_layout_passes=False),
    )
    def gather(qc_hbm, cc_hbm, actr_hbm, cctr_hbm, hi_hbm, wi_hbm,
               qh_out, cw_out, dif_out,
               iah, iaw, ibh, ibw, bqa, bca, bqb, bcb, dbuf, dih, diw,
               actr_v, cctr_v, gsa, gsb, ssa, ssb):
        wid = lax.axis_index("c") * ns + lax.axis_index("s")
        wbase = pl.multiple_of(wid * ew, 8)
        pltpu.sync_copy(actr_hbm, actr_v)
        pltpu.sync_copy(cctr_hbm, cctr_v)
        iota = lax.iota(jnp.int32, 16)

        def zero_row(i, _):
            for k in range(8):
                dbuf[i, pl.ds(k * 16, 16)] = jnp.zeros((16,), jnp.float32)
            return ()

        lax.fori_loop(0, rchunk, zero_row, (), unroll=False)

        @pl.when(wid < nblk)
        def _():
            dbase = pl.multiple_of(wid * be, 8)
            pltpu.sync_copy(hi_hbm.at[pl.ds(dbase, be)], dih)
            pltpu.sync_copy(wi_hbm.at[pl.ds(dbase, be)], diw)
            for rc in range(rpb // rchunk):
                for j in range(8):
                    for g in range(rchunk // 16):
                        off = j * rpb + rc * rchunk + g * 16
                        rh = dih[pl.ds(off, 16)] * 2
                        rw = diw[pl.ds(off, 16)] * 2
                        d0 = (plsc.load_gather(actr_v, [rh])
                              - plsc.load_gather(cctr_v, [rw]))
                        d1 = (plsc.load_gather(actr_v, [rh + 1])
                              - plsc.load_gather(cctr_v, [rw + 1]))
                        rows = g * 16 + iota
                        cols = jnp.full((16,), 16 * j, jnp.int32)
                        plsc.store_scatter(dbuf, [rows, cols], d0)
                        plsc.store_scatter(dbuf, [rows, cols + 1], d1)
                rowb = pl.multiple_of(wid * rpb + rc * rchunk, 8)
                pltpu.sync_copy(dbuf, dif_out.at[pl.ds(rowb, rchunk)])

        def fire_stores(base, bq, bc, sem):
            pltpu.async_copy(bq, qh_out.at[pl.ds(base, ch)], sem)
            pltpu.async_copy(bc, cw_out.at[pl.ds(base, ch)], sem)

        def drain_stores(base, bq, bc, sem):
            pltpu.make_async_copy(bq, qh_out.at[pl.ds(base, ch)], sem).wait()
            pltpu.make_async_copy(bc, cw_out.at[pl.ds(base, ch)], sem).wait()

        def drain_gathers(base, bq, bc, sem):
            pltpu.make_async_copy(qh_out.at[pl.ds(base, ch)], bq, sem).wait()
            pltpu.make_async_copy(cw_out.at[pl.ds(base, ch)], bc, sem).wait()

        base0 = pl.multiple_of(wbase, 8)
        pltpu.sync_copy(hi_hbm.at[pl.ds(base0, ch)], iah)
        pltpu.sync_copy(wi_hbm.at[pl.ds(base0, ch)], iaw)
        pltpu.async_copy(qc_hbm.at[iah], bqa, gsa)
        pltpu.async_copy(cc_hbm.at[iaw], bca, gsa)

        def pair(t, _):
            base_a = pl.multiple_of(wbase + 2 * t * ch, 8)
            base_b = pl.multiple_of(base_a + ch, 8)
            base_a2 = pl.multiple_of(base_b + ch, 8)
            pltpu.sync_copy(hi_hbm.at[pl.ds(base_b, ch)], ibh)
            pltpu.sync_copy(wi_hbm.at[pl.ds(base_b, ch)], ibw)

            @pl.when(t > 0)
            def _():
                drain_stores(base_b - 2 * ch, bqb, bcb, ssb)

            g1 = pltpu.async_copy(qc_hbm.at[ibh], bqb, gsb)
            g2 = pltpu.async_copy(cc_hbm.at[ibw], bcb, gsb)
            drain_gathers(base_a, bqa, bca, gsa)
            sa1 = pltpu.async_copy(bqa, qh_out.at[pl.ds(base_a, ch)], ssa)
            sa2 = pltpu.async_copy(bca, cw_out.at[pl.ds(base_a, ch)], ssa)
            g1.wait()
            g2.wait()
            fire_stores(base_b, bqb, bcb, ssb)
            pltpu.sync_copy(hi_hbm.at[pl.ds(base_a2, ch)], iah)
            pltpu.sync_copy(wi_hbm.at[pl.ds(base_a2, ch)], iaw)
            sa1.wait()
            sa2.wait()
            pltpu.async_copy(qc_hbm.at[iah], bqa, gsa)
            pltpu.async_copy(cc_hbm.at[iaw], bca, gsa)
            return ()

        lax.fori_loop(0, npairs, pair, (), unroll=False)
        base_l = pl.multiple_of(wbase + (nchunks - 1) * ch, 8)
        drain_stores(base_l - ch, bqb, bcb, ssb)
        drain_gathers(base_l, bqa, bca, gsa)
        fire_stores(base_l, bqa, bca, ssa)
        drain_stores(base_l, bqa, bca, ssa)

    return gather


def _edge_body(dif_ref, qh_ref, cw_ref, w1p_ref, b1_ref, w2_ref,
               g2_ref, b2_ref, w1d_ref, g1_ref, b1c_ref, wc2_ref, c_ref):
    dp8 = jnp.dot(dif_ref[...], w1p_ref[...],
                  preferred_element_type=jnp.float32)
    dpre = jnp.concatenate([dp8[:, j * 128:(j + 1) * 128] for j in range(8)],
                           axis=0)
    d = jax.nn.relu(dpre + b1_ref[...])
    d = jax.nn.relu(_gn(jnp.dot(d, w2_ref[...],
                                preferred_element_type=jnp.float32),
                        g2_ref[...], b2_ref[...]))
    cpre = (jnp.dot(d, w1d_ref[...], preferred_element_type=jnp.float32)
            + qh_ref[...] + cw_ref[...])
    c = jax.nn.relu(_gn(cpre, g1_ref[...], b1c_ref[...]))
    c_ref[...] = jnp.dot(c, wc2_ref[...], preferred_element_type=jnp.float32)


def _edge_mlp(dif, qh, cw, w1p, b1, w2, g2, b2, w1d, g1, b1c, wc2, be):
    e, d = qh.shape
    grid = e // be
    row = pl.BlockSpec((be, d), lambda i: (i, 0))
    row16 = pl.BlockSpec((be // 8, 128), lambda i: (i, 0))
    mat = pl.BlockSpec((d, d), lambda i: (0, 0))
    mat16 = pl.BlockSpec((128, 8 * d), lambda i: (0, 0))
    vec = pl.BlockSpec((1, d), lambda i: (0, 0))
    return pl.pallas_call(
        _edge_body,
        grid=(grid,),
        in_specs=[row16, row, row, mat16, vec, mat, vec, vec, mat,
                  vec, vec, mat],
        out_specs=row,
        out_shape=jax.ShapeDtypeStruct((e, d), jnp.float32),
    )(dif, qh, cw, w1p, b1.reshape(1, d), w2, g2.reshape(1, d),
      b2.reshape(1, d), w1d, g1.reshape(1, d), b1c.reshape(1, d), wc2)


def _make_scatter(n, e, d, nc, ns, ch, nsl):
    esl = e // nsl
    ew = esl // (nc * ns)
    nchunks = ew // ch
    rpt = 1000
    mesh = plsc.VectorSubcoreMesh(core_axis_name="c", subcore_axis_name="s")

    assert nchunks % 2 == 1, "pipelined schedule assumes an odd chunk count"
    npairs = (nchunks - 1) // 2

    @functools.partial(
        pl.kernel,
        mesh=mesh,
        out_type=jax.ShapeDtypeStruct((nc * n, d), jnp.float32),
        scratch_types=[
            pltpu.VMEM_SHARED((n, d), jnp.float32),
            pltpu.VMEM((ch,), jnp.int32),
            pltpu.VMEM((ch,), jnp.int32),
            pltpu.VMEM((ch, d), jnp.float32),
            pltpu.VMEM((ch, d), jnp.float32),
            pltpu.SemaphoreType.DMA,
            pltpu.SemaphoreType.DMA,
        ],
        compiler_params=pltpu.CompilerParams(needs_layout_passes=False),
    )
    def scatter(*refs):
        c_list = refs[:nsl]
        hi_hbm = refs[nsl]
        zeros_hbm = refs[nsl + 1]
        part_out = refs[nsl + 2]
        acc, ia, ib, bva, bvb, sla, slb = refs[nsl + 3:]
        cid = lax.axis_index("c")
        sid = lax.axis_index("s")
        wid = cid * ns + sid

        @pl.when(sid == 0)
        def _():
            pltpu.sync_copy(zeros_hbm, acc)

        plsc.subcore_barrier()

        for s in range(nsl):
            c_hbm = c_list[s]
            wbase = pl.multiple_of(wid * ew, 8)
            hbase = pl.multiple_of(s * esl + wid * ew, 8)

            def fire_loads(base, ix, bv, sem):
                pltpu.async_copy(hi_hbm.at[pl.ds(base + (hbase - wbase), ch)],
                                 ix, sem)
                pltpu.async_copy(c_hbm.at[pl.ds(base, ch)], bv, sem)

            def drain_loads(base, ix, bv, sem):
                pltpu.make_async_copy(
                    hi_hbm.at[pl.ds(base + (hbase - wbase), ch)],
                    ix, sem).wait()
                pltpu.make_async_copy(c_hbm.at[pl.ds(base, ch)], bv,
                                      sem).wait()

            fire_loads(wbase, ia, bva, sla)

            def pair(t, _):
                base_a = pl.multiple_of(wbase + 2 * t * ch, 8)
                base_b = pl.multiple_of(base_a + ch, 8)
                base_a2 = pl.multiple_of(base_b + ch, 8)
                fire_loads(base_b, ib, bvb, slb)
                drain_loads(base_a, ia, bva, sla)
                pltpu.sync_copy(bva, acc.at[ia], add=True)
                fire_loads(base_a2, ia, bva, sla)
                drain_loads(base_b, ib, bvb, slb)
                pltpu.sync_copy(bvb, acc.at[ib], add=True)
                return ()

            lax.fori_loop(0, npairs, pair, (), unroll=False)
            base_l = pl.multiple_of(wbase + (nchunks - 1) * ch, 8)
            drain_loads(base_l, ia, bva, sla)
            pltpu.sync_copy(bva, acc.at[ia], add=True)
        plsc.subcore_barrier()

        @pl.when(sid < n // rpt)
        def _():
            rbase = pl.multiple_of(sid * rpt, 8)
            pltpu.sync_copy(acc.at[pl.ds(rbase, rpt)],
                            part_out.at[pl.ds(cid * n + rbase, rpt)])

    return scatter


def _final_body(a0_ref, p0_ref, p1_ref, res_ref, ng_ref, nb_ref, lw_ref,
                lg_ref, lb_ref, out_ref):
    a = a0_ref[...] + p0_ref[...] + p1_ref[...]
    a = jax.nn.relu(_gn(a, ng_ref[...], nb_ref[...]))
    a = _gn(jnp.dot(a, lw_ref[...], preferred_element_type=jnp.float32),
            lg_ref[...], lb_ref[...])
    out_ref[...] = jax.nn.relu(a + res_ref[...])


def _node_final(a0, p0, p1, res, norm_g, norm_b, lin_w, lin_g, lin_b, bn):
    n, d = a0.shape
    grid = n // bn
    row = pl.BlockSpec((bn, d), lambda i: (i, 0))
    mat = pl.BlockSpec((d, d), lambda i: (0, 0))
    vec = pl.BlockSpec((1, d), lambda i: (0, 0))
    return pl.pallas_call(
        _final_body,
        grid=(grid,),
        in_specs=[row, row, row, row, vec, vec, mat, vec, vec],
        out_specs=row,
        out_shape=jax.ShapeDtypeStruct((n, d), jnp.float32),
    )(a0, p0, p1, res, norm_g.reshape(1, d), norm_b.reshape(1, d), lin_w,
      lin_g.reshape(1, d), lin_b.reshape(1, d))


def kernel(agts, agt_ctrs, ctx, ctx_ctrs, edge_index, dist_w1, dist_b1,
           dist_w2, dist_g2, dist_b2, query_w, query_g, query_b, ctx_w1,
           ctx_g1, ctx_b1, ctx_w2, agt_w, norm_g, norm_b, lin_w, lin_g,
           lin_b):
    n, d = agts.shape
    e = edge_index.shape[1]
    ncc = agt_ctrs.shape[1]
    nc, ns = 2, 16

    hi = edge_index[0]
    wi = edge_index[1]
    w1d, w1q, w1c = ctx_w1[:d], ctx_w1[d:2 * d], ctx_w1[2 * d:]

    actr_f = agt_ctrs.reshape(-1)
    cctr_f = ctx_ctrs.reshape(-1)
    w1p16 = jnp.zeros((16, d), jnp.float32).at[:ncc].set(dist_w1)
    w1p = (jnp.eye(8, dtype=jnp.float32)[:, None, :, None]
           * w1p16[None, :, None, :]).reshape(128, 8 * d)
    zeros_nd = jnp.zeros((n, d), jnp.float32)

    qc, cc, a0 = _node_pre(agts, ctx, query_w, query_g, query_b, w1q, w1c,
                           agt_w, bn=2000)

    nsl = 5
    esl = e // nsl
    gather = _make_gather(n, esl, d, nc, ns, ch=80, be=3200)
    edge = functools.partial(_edge_mlp, be=3200)
    cs = []
    for s in range(nsl):
        qh, cw, dif_p = gather(qc, cc, actr_f, cctr_f,
                               hi[s * esl:(s + 1) * esl],
                               wi[s * esl:(s + 1) * esl])
        cs.append(edge(dif_p, qh, cw, w1p, dist_b1,
                       dist_w2, dist_g2, dist_b2, w1d, ctx_g1, ctx_b1,
                       ctx_w2))
    scatter = _make_scatter(n, e, d, nc, ns, ch=80, nsl=nsl)
    parts = scatter(*cs, hi, zeros_nd)
    p0 = parts[:n]
    p1 = parts[n:]

    return _node_final(a0, p0, p1, agts, norm_g, norm_b, lin_w, lin_g, lin_b,
                       bn=2000)


# speedup vs baseline: 1.0372x; 1.0372x over previous
import functools

import jax
import jax.numpy as jnp
from jax import lax
from jax.experimental import pallas as pl
from jax.experimental.pallas import tpu as pltpu
from jax.experimental.pallas import tpu_sc as plsc


def _gn(x, g, b, eps=1e-5):
    mu = jnp.mean(x, axis=-1, keepdims=True)
    var = jnp.mean((x - mu) ** 2, axis=-1, keepdims=True)
    return (x - mu) / jnp.sqrt(var + eps) * g + b


def _pre_body(agts_ref, ctx_ref, qw_ref, qg_ref, qb_ref, w1q_ref, w1c_ref,
              aw_ref, qc_ref, cc_ref, a0_ref):
    x = agts_ref[...]
    q = jax.nn.relu(_gn(jnp.dot(x, qw_ref[...],
                                preferred_element_type=jnp.float32),
                        qg_ref[...], qb_ref[...]))
    qc_ref[...] = jnp.dot(q, w1q_ref[...], preferred_element_type=jnp.float32)
    cc_ref[...] = jnp.dot(ctx_ref[...], w1c_ref[...],
                          preferred_element_type=jnp.float32)
    a0_ref[...] = jnp.dot(x, aw_ref[...], preferred_element_type=jnp.float32)


def _node_pre(agts, ctx, query_w, query_g, query_b, w1q, w1c, agt_w, bn):
    n, d = agts.shape
    grid = n // bn
    row = pl.BlockSpec((bn, d), lambda i: (i, 0))
    mat = pl.BlockSpec((d, d), lambda i: (0, 0))
    vec = pl.BlockSpec((1, d), lambda i: (0, 0))
    out = jax.ShapeDtypeStruct((n, d), jnp.float32)
    outh = jax.ShapeDtypeStruct((n, d), jnp.bfloat16)
    return pl.pallas_call(
        _pre_body,
        grid=(grid,),
        in_specs=[row, row, mat, vec, vec, mat, mat, mat],
        out_specs=[row, row, row],
        out_shape=[out, out, out],
    )(agts, ctx, query_w, query_g.reshape(1, d), query_b.reshape(1, d),
      w1q, w1c, agt_w)


def _make_gather(n, e, d, nc, ns, ch, be):
    ew = e // (nc * ns)
    nchunks = ew // ch
    assert nchunks % 2 == 1, "pipelined schedule assumes an odd chunk count"
    npairs = (nchunks - 1) // 2
    nblk = e // be
    rpb = be // 8
    rchunk = 80
    mesh = plsc.VectorSubcoreMesh(core_axis_name="c", subcore_axis_name="s")

    @functools.partial(
        pl.kernel,
        mesh=mesh,
        out_type=[
            jax.ShapeDtypeStruct((e, d), jnp.float32),
            jax.ShapeDtypeStruct((e // 8, 128), jnp.float32),
        ],
        scratch_types=[
            pltpu.VMEM((ch,), jnp.int32),
            pltpu.VMEM((ch,), jnp.int32),
            pltpu.VMEM((ch,), jnp.int32),
            pltpu.VMEM((ch,), jnp.int32),
            pltpu.VMEM((ch, d), jnp.float32),
            pltpu.VMEM((ch, d), jnp.float32),
            pltpu.VMEM((ch, d), jnp.float32),
            pltpu.VMEM((ch, d), jnp.float32),
            pltpu.VMEM((rchunk, 128), jnp.float32),
            pltpu.VMEM((be,), jnp.int32),
            pltpu.VMEM((be,), jnp.int32),
            pltpu.VMEM((2 * n,), jnp.float32),
            pltpu.VMEM((2 * n,), jnp.float32),
            pltpu.SemaphoreType.DMA,
            pltpu.SemaphoreType.DMA,
            pltpu.SemaphoreType.DMA,
            pltpu.SemaphoreType.DMA,
        ],
        compiler_params=pltpu.CompilerParams(needs_layout_passes=False),
    )
    def gather(qc_hbm, cc_hbm, actr_hbm, cctr_hbm, hi_hbm, wi_hbm,
               qs_out, dif_out,
               iah, iaw, ibh, ibw, bqa, bca, bqb, bcb, dbuf, dih, diw,
               actr_v, cctr_v, gsa, gsb, ssa, ssb):
        wid = lax.axis_index("c") * ns + lax.axis_index("s")
        wbase = pl.multiple_of(wid * ew, 8)
        iota = lax.iota(jnp.int32, 16)

        base0 = pl.multiple_of(wbase, 8)
        pltpu.sync_copy(hi_hbm.at[pl.ds(base0, ch)], iah)
        pltpu.sync_copy(wi_hbm.at[pl.ds(base0, ch)], iaw)
        pltpu.async_copy(qc_hbm.at[iah], bqa, gsa)
        pltpu.async_copy(cc_hbm.at[iaw], bca, gsa)
        pltpu.sync_copy(actr_hbm, actr_v)
        pltpu.sync_copy(cctr_hbm, cctr_v)

        def zero_row(i, _):
            for k in range(8):
                dbuf[i, pl.ds(k * 16, 16)] = jnp.zeros((16,), jnp.float32)
            return ()

        lax.fori_loop(0, rchunk, zero_row, (), unroll=False)

        @pl.when(wid < nblk)
        def _():
            dbase = pl.multiple_of(wid * be, 8)
            pltpu.sync_copy(hi_hbm.at[pl.ds(dbase, be)], dih)
            pltpu.sync_copy(wi_hbm.at[pl.ds(dbase, be)], diw)
            for rc in range(rpb // rchunk):
                for j in range(8):
                    for g in range(rchunk // 16):
                        off = j * rpb + rc * rchunk + g * 16
                        rh = dih[pl.ds(off, 16)] * 2
                        rw = diw[pl.ds(off, 16)] * 2
                        d0 = (plsc.load_gather(actr_v, [rh])
                              - plsc.load_gather(cctr_v, [rw]))
                        d1 = (plsc.load_gather(actr_v, [rh + 1])
                              - plsc.load_gather(cctr_v, [rw + 1]))
                        rows = g * 16 + iota
                        cols = jnp.full((16,), 16 * j, jnp.int32)
                        plsc.store_scatter(dbuf, [rows, cols], d0)
                        plsc.store_scatter(dbuf, [rows, cols + 1], d1)
                rowb = pl.multiple_of(wid * rpb + rc * rchunk, 8)
                pltpu.sync_copy(dbuf, dif_out.at[pl.ds(rowb, rchunk)])

        def vsum(bq, bc):
            def row(r, _):
                for k in range(8):
                    sl = pl.ds(k * 16, 16)
                    bq[r, sl] = bq[r, sl] + bc[r, sl]
                return ()

            lax.fori_loop(0, ch, row, (), unroll=False)

        def drain_store(base, bq, sem):
            pltpu.make_async_copy(bq, qs_out.at[pl.ds(base, ch)], sem).wait()

        def drain_gathers(base, bq, bc, sem):
            pltpu.make_async_copy(qs_out.at[pl.ds(base, ch)], bq, sem).wait()
            pltpu.make_async_copy(qs_out.at[pl.ds(base, ch)], bc, sem).wait()

        def pair(t, _):
            base_a = pl.multiple_of(wbase + 2 * t * ch, 8)
            base_b = pl.multiple_of(base_a + ch, 8)
            base_a2 = pl.multiple_of(base_b + ch, 8)
            pltpu.sync_copy(hi_hbm.at[pl.ds(base_b, ch)], ibh)
            pltpu.sync_copy(wi_hbm.at[pl.ds(base_b, ch)], ibw)

            @pl.when(t > 0)
            def _():
                drain_store(base_b - 2 * ch, bqb, ssb)

            g1 = pltpu.async_copy(qc_hbm.at[ibh], bqb, gsb)
            g2 = pltpu.async_copy(cc_hbm.at[ibw], bcb, gsb)
            drain_gathers(base_a, bqa, bca, gsa)
            vsum(bqa, bca)
            sa1 = pltpu.async_copy(bqa, qs_out.at[pl.ds(base_a, ch)], ssa)
            g1.wait()
            g2.wait()
            vsum(bqb, bcb)
            pltpu.async_copy(bqb, qs_out.at[pl.ds(base_b, ch)], ssb)
            pltpu.sync_copy(hi_hbm.at[pl.ds(base_a2, ch)], iah)
            pltpu.sync_copy(wi_hbm.at[pl.ds(base_a2, ch)], iaw)
            sa1.wait()
            pltpu.async_copy(qc_hbm.at[iah], bqa, gsa)
            pltpu.async_copy(cc_hbm.at[iaw], bca, gsa)
            return ()

        lax.fori_loop(0, npairs, pair, (), unroll=False)
        base_l = pl.multiple_of(wbase + (nchunks - 1) * ch, 8)
        drain_store(base_l - ch, bqb, ssb)
        drain_gathers(base_l, bqa, bca, gsa)
        vsum(bqa, bca)
        pltpu.sync_copy(bqa, qs_out.at[pl.ds(base_l, ch)])

    return gather


def _edge_body(dif_ref, qs_ref, w1p_ref, b1_ref, w2_ref,
               g2_ref, b2_ref, w1d_ref, g1_ref, b1c_ref, wc2_ref, c_ref):
    dp8 = jnp.dot(dif_ref[...], w1p_ref[...],
                  preferred_element_type=jnp.float32)
    dpre = jnp.concatenate([dp8[:, j * 128:(j + 1) * 128] for j in range(8)],
                           axis=0)
    d = jax.nn.relu(dpre + b1_ref[...])
    d = jax.nn.relu(_gn(jnp.dot(d, w2_ref[...],
                                preferred_element_type=jnp.float32),
                        g2_ref[...], b2_ref[...]))
    cpre = (jnp.dot(d, w1d_ref[...], preferred_element_type=jnp.float32)
            + qs_ref[...])
    c = jax.nn.relu(_gn(cpre, g1_ref[...], b1c_ref[...]))
    c_ref[...] = jnp.dot(c, wc2_ref[...], preferred_element_type=jnp.float32)


def _edge_mlp(dif, qs, w1p, b1, w2, g2, b2, w1d, g1, b1c, wc2, be):
    e, d = qs.shape
    grid = e // be
    row = pl.BlockSpec((be, d), lambda i: (i, 0))
    row16 = pl.BlockSpec((be // 8, 128), lambda i: (i, 0))
    mat = pl.BlockSpec((d, d), lambda i: (0, 0))
    mat16 = pl.BlockSpec((128, 8 * d), lambda i: (0, 0))
    vec = pl.BlockSpec((1, d), lambda i: (0, 0))
    return pl.pallas_call(
        _edge_body,
        grid=(grid,),
        in_specs=[row16, row, mat16, vec, mat, vec, vec, mat,
                  vec, vec, mat],
        out_specs=row,
        out_shape=jax.ShapeDtypeStruct((e, d), jnp.float32),
    )(dif, qs, w1p, b1.reshape(1, d), w2, g2.reshape(1, d),
      b2.reshape(1, d), w1d, g1.reshape(1, d), b1c.reshape(1, d), wc2)


def _make_scatter(n, e, d, nc, ns, ch, nsl):
    esl = e // nsl
    ew = esl // (nc * ns)
    nchunks = ew // ch
    rpt = 1000
    mesh = plsc.VectorSubcoreMesh(core_axis_name="c", subcore_axis_name="s")

    assert nchunks % 2 == 1, "pipelined schedule assumes an odd chunk count"
    npairs = (nchunks - 1) // 2

    @functools.partial(
        pl.kernel,
        mesh=mesh,
        out_type=jax.ShapeDtypeStruct((nc * n, d), jnp.float32),
        scratch_types=[
            pltpu.VMEM_SHARED((n, d), jnp.float32),
            pltpu.VMEM((ch,), jnp.int32),
            pltpu.VMEM((ch,), jnp.int32),
            pltpu.VMEM((ch, d), jnp.float32),
            pltpu.VMEM((ch, d), jnp.float32),
            pltpu.SemaphoreType.DMA,
            pltpu.SemaphoreType.DMA,
        ],
        compiler_params=pltpu.CompilerParams(needs_layout_passes=False),
    )
    def scatter(*refs):
        c_list = refs[:nsl]
        hi_hbm = refs[nsl]
        zeros_hbm = refs[nsl + 1]
        part_out = refs[nsl + 2]
        acc, ia, ib, bva, bvb, sla, slb = refs[nsl + 3:]
        cid = lax.axis_index("c")
        sid = lax.axis_index("s")
        wid = cid * ns + sid

        @pl.when(sid == 0)
        def _():
            pltpu.sync_copy(zeros_hbm, acc)

        plsc.subcore_barrier()

        for s in range(nsl):
            c_hbm = c_list[s]
            wbase = pl.multiple_of(wid * ew, 8)
            hbase = pl.multiple_of(s * esl + wid * ew, 8)

            def fire_loads(base, ix, bv, sem):
                pltpu.async_copy(hi_hbm.at[pl.ds(base + (hbase - wbase), ch)],
                                 ix, sem)
                pltpu.async_copy(c_hbm.at[pl.ds(base, ch)], bv, sem)

            def drain_loads(base, ix, bv, sem):
                pltpu.make_async_copy(
                    hi_hbm.at[pl.ds(base + (hbase - wbase), ch)],
                    ix, sem).wait()
                pltpu.make_async_copy(c_hbm.at[pl.ds(base, ch)], bv,
                                      sem).wait()

            fire_loads(wbase, ia, bva, sla)

            def pair(t, _):
                base_a = pl.multiple_of(wbase + 2 * t * ch, 8)
                base_b = pl.multiple_of(base_a + ch, 8)
                base_a2 = pl.multiple_of(base_b + ch, 8)
                fire_loads(base_b, ib, bvb, slb)
                drain_loads(base_a, ia, bva, sla)
                pltpu.sync_copy(bva, acc.at[ia], add=True)
                fire_loads(base_a2, ia, bva, sla)
                drain_loads(base_b, ib, bvb, slb)
                pltpu.sync_copy(bvb, acc.at[ib], add=True)
                return ()

            lax.fori_loop(0, npairs, pair, (), unroll=False)
            base_l = pl.multiple_of(wbase + (nchunks - 1) * ch, 8)
            drain_loads(base_l, ia, bva, sla)
            pltpu.sync_copy(bva, acc.at[ia], add=True)
        plsc.subcore_barrier()

        @pl.when(sid < n // rpt)
        def _():
            rbase = pl.multiple_of(sid * rpt, 8)
            pltpu.sync_copy(acc.at[pl.ds(rbase, rpt)],
                            part_out.at[pl.ds(cid * n + rbase, rpt)])

    return scatter


def _final_body(a0_ref, p0_ref, p1_ref, res_ref, ng_ref, nb_ref, lw_ref,
                lg_ref, lb_ref, out_ref):
    a = a0_ref[...] + p0_ref[...] + p1_ref[...]
    a = jax.nn.relu(_gn(a, ng_ref[...], nb_ref[...]))
    a = _gn(jnp.dot(a, lw_ref[...], preferred_element_type=jnp.float32),
            lg_ref[...], lb_ref[...])
    out_ref[...] = jax.nn.relu(a + res_ref[...])


def _node_final(a0, p0, p1, res, norm_g, norm_b, lin_w, lin_g, lin_b, bn):
    n, d = a0.shape
    grid = n // bn
    row = pl.BlockSpec((bn, d), lambda i: (i, 0))
    mat = pl.BlockSpec((d, d), lambda i: (0, 0))
    vec = pl.BlockSpec((1, d), lambda i: (0, 0))
    return pl.pallas_call(
        _final_body,
        grid=(grid,),
        in_specs=[row, row, row, row, vec, vec, mat, vec, vec],
        out_specs=row,
        out_shape=jax.ShapeDtypeStruct((n, d), jnp.float32),
    )(a0, p0, p1, res, norm_g.reshape(1, d), norm_b.reshape(1, d), lin_w,
      lin_g.reshape(1, d), lin_b.reshape(1, d))


def kernel(agts, agt_ctrs, ctx, ctx_ctrs, edge_index, dist_w1, dist_b1,
           dist_w2, dist_g2, dist_b2, query_w, query_g, query_b, ctx_w1,
           ctx_g1, ctx_b1, ctx_w2, agt_w, norm_g, norm_b, lin_w, lin_g,
           lin_b):
    n, d = agts.shape
    e = edge_index.shape[1]
    ncc = agt_ctrs.shape[1]
    nc, ns = 2, 16

    hi = edge_index[0]
    wi = edge_index[1]
    w1d, w1q, w1c = ctx_w1[:d], ctx_w1[d:2 * d], ctx_w1[2 * d:]

    actr_f = agt_ctrs.reshape(-1)
    cctr_f = ctx_ctrs.reshape(-1)
    w1p16 = jnp.zeros((16, d), jnp.float32).at[:ncc].set(dist_w1)
    w1p = (jnp.eye(8, dtype=jnp.float32)[:, None, :, None]
           * w1p16[None, :, None, :]).reshape(128, 8 * d)
    zeros_nd = jnp.zeros((n, d), jnp.float32)

    qc, cc, a0 = _node_pre(agts, ctx, query_w, query_g, query_b, w1q, w1c,
                           agt_w, bn=2000)

    nsl = 5
    esl = e // nsl
    gather = _make_gather(n, esl, d, nc, ns, ch=80, be=3200)
    edge = functools.partial(_edge_mlp, be=3200)
    cs = []
    for s in range(nsl):
        qs, dif_p = gather(qc, cc, actr_f, cctr_f,
                           hi[s * esl:(s + 1) * esl],
                           wi[s * esl:(s + 1) * esl])
        cs.append(edge(dif_p, qs, w1p, dist_b1,
                       dist_w2, dist_g2, dist_b2, w1d, ctx_g1, ctx_b1,
                       ctx_w2))
    scatter = _make_scatter(n, e, d, nc, ns, ch=80, nsl=nsl)
    parts = scatter(*cs, hi, zeros_nd)
    p0 = parts[:n]
    p1 = parts[n:]

    return _node_final(a0, p0, p1, agts, norm_g, norm_b, lin_w, lin_g, lin_b,
                       bn=2000)


# speedup vs baseline: 1.0794x; 1.0406x over previous
import functools

import jax
import jax.numpy as jnp
from jax import lax
from jax.experimental import pallas as pl
from jax.experimental.pallas import tpu as pltpu
from jax.experimental.pallas import tpu_sc as plsc


def _gn(x, g, b, eps=1e-5):
    mu = jnp.mean(x, axis=-1, keepdims=True)
    var = jnp.mean((x - mu) ** 2, axis=-1, keepdims=True)
    return (x - mu) / jnp.sqrt(var + eps) * g + b


def _pre_body(agts_ref, ctx_ref, qw_ref, qg_ref, qb_ref, w1q_ref, w1c_ref,
              aw_ref, qc_ref, cc_ref, a0_ref):
    x = agts_ref[...]
    q = jax.nn.relu(_gn(jnp.dot(x, qw_ref[...],
                                preferred_element_type=jnp.float32),
                        qg_ref[...], qb_ref[...]))
    qc_ref[...] = jnp.dot(q, w1q_ref[...], preferred_element_type=jnp.float32)
    cc_ref[...] = jnp.dot(ctx_ref[...], w1c_ref[...],
                          preferred_element_type=jnp.float32)
    a0_ref[...] = jnp.dot(x, aw_ref[...], preferred_element_type=jnp.float32)


def _node_pre(agts, ctx, query_w, query_g, query_b, w1q, w1c, agt_w, bn):
    n, d = agts.shape
    grid = n // bn
    row = pl.BlockSpec((bn, d), lambda i: (i, 0))
    mat = pl.BlockSpec((d, d), lambda i: (0, 0))
    vec = pl.BlockSpec((1, d), lambda i: (0, 0))
    out = jax.ShapeDtypeStruct((n, d), jnp.float32)
    outh = jax.ShapeDtypeStruct((n, d), jnp.bfloat16)
    return pl.pallas_call(
        _pre_body,
        grid=(grid,),
        in_specs=[row, row, mat, vec, vec, mat, mat, mat],
        out_specs=[row, row, row],
        out_shape=[out, out, out],
    )(agts, ctx, query_w, query_g.reshape(1, d), query_b.reshape(1, d),
      w1q, w1c, agt_w)


def _make_gather(n, e, d, nc, ns, ch, be):
    ew = e // (nc * ns)
    nchunks = ew // ch
    assert nchunks % 2 == 1, "pipelined schedule assumes an odd chunk count"
    npairs = (nchunks - 1) // 2
    nblk = e // be
    rpb = be // 8
    rchunk = 80
    mesh = plsc.VectorSubcoreMesh(core_axis_name="c", subcore_axis_name="s")

    @functools.partial(
        pl.kernel,
        mesh=mesh,
        out_type=[
            jax.ShapeDtypeStruct((e, d), jnp.float32),
            jax.ShapeDtypeStruct((e // 8, 128), jnp.float32),
        ],
        scratch_types=[
            pltpu.VMEM((ch,), jnp.int32),
            pltpu.VMEM((ch,), jnp.int32),
            pltpu.VMEM((ch,), jnp.int32),
            pltpu.VMEM((ch,), jnp.int32),
            pltpu.VMEM((ch, d), jnp.float32),
            pltpu.VMEM((ch, d), jnp.float32),
            pltpu.VMEM((ch, d), jnp.float32),
            pltpu.VMEM((ch, d), jnp.float32),
            pltpu.VMEM((rchunk, 128), jnp.float32),
            pltpu.VMEM((be,), jnp.int32),
            pltpu.VMEM((be,), jnp.int32),
            pltpu.VMEM((2 * n,), jnp.float32),
            pltpu.VMEM((2 * n,), jnp.float32),
            pltpu.SemaphoreType.DMA,
            pltpu.SemaphoreType.DMA,
            pltpu.SemaphoreType.DMA,
            pltpu.SemaphoreType.DMA,
        ],
        compiler_params=pltpu.CompilerParams(needs_layout_passes=False),
    )
    def gather(qc_hbm, cc_hbm, actr_hbm, cctr_hbm, hi_hbm, wi_hbm,
               qs_out, dif_out,
               iah, iaw, ibh, ibw, bqa, bca, bqb, bcb, dbuf, dih, diw,
               actr_v, cctr_v, gsa, gsb, ssa, ssb):
        wid = lax.axis_index("c") * ns + lax.axis_index("s")
        wbase = pl.multiple_of(wid * ew, 8)
        iota = lax.iota(jnp.int32, 16)

        base0 = pl.multiple_of(wbase, 8)
        pltpu.sync_copy(hi_hbm.at[pl.ds(base0, ch)], iah)
        pltpu.sync_copy(wi_hbm.at[pl.ds(base0, ch)], iaw)
        pltpu.async_copy(qc_hbm.at[iah], bqa, gsa)
        pltpu.async_copy(cc_hbm.at[iaw], bca, gsa)
        pltpu.sync_copy(actr_hbm, actr_v)
        pltpu.sync_copy(cctr_hbm, cctr_v)

        def zero_row(i, _):
            for k in range(8):
                dbuf[i, pl.ds(k * 16, 16)] = jnp.zeros((16,), jnp.float32)
            return ()

        lax.fori_loop(0, rchunk, zero_row, (), unroll=False)

        @pl.when(wid < nblk)
        def _():
            dbase = pl.multiple_of(wid * be, 8)
            pltpu.sync_copy(hi_hbm.at[pl.ds(dbase, be)], dih)
            pltpu.sync_copy(wi_hbm.at[pl.ds(dbase, be)], diw)
            for rc in range(rpb // rchunk):
                for j in range(8):
                    for g in range(rchunk // 16):
                        off = j * rpb + rc * rchunk + g * 16
                        rh = dih[pl.ds(off, 16)] * 2
                        rw = diw[pl.ds(off, 16)] * 2
                        d0 = (plsc.load_gather(actr_v, [rh])
                              - plsc.load_gather(cctr_v, [rw]))
                        d1 = (plsc.load_gather(actr_v, [rh + 1])
                              - plsc.load_gather(cctr_v, [rw + 1]))
                        rows = g * 16 + iota
                        cols = jnp.full((16,), 16 * j, jnp.int32)
                        plsc.store_scatter(dbuf, [rows, cols], d0)
                        plsc.store_scatter(dbuf, [rows, cols + 1], d1)
                rowb = pl.multiple_of(wid * rpb + rc * rchunk, 8)
                pltpu.sync_copy(dbuf, dif_out.at[pl.ds(rowb, rchunk)])

        def vsum(bq, bc):
            def row(r, _):
                for k in range(8):
                    sl = pl.ds(k * 16, 16)
                    bq[r, sl] = bq[r, sl] + bc[r, sl]
                return ()

            lax.fori_loop(0, ch, row, (), unroll=False)

        def drain_store(base, bq, sem):
            pltpu.make_async_copy(bq, qs_out.at[pl.ds(base, ch)], sem).wait()

        def drain_gathers(base, bq, bc, sem):
            pltpu.make_async_copy(qs_out.at[pl.ds(base, ch)], bq, sem).wait()
            pltpu.make_async_copy(qs_out.at[pl.ds(base, ch)], bc, sem).wait()

        def pair(t, _):
            base_a = pl.multiple_of(wbase + 2 * t * ch, 8)
            base_b = pl.multiple_of(base_a + ch, 8)
            base_a2 = pl.multiple_of(base_b + ch, 8)
            pltpu.sync_copy(hi_hbm.at[pl.ds(base_b, ch)], ibh)
            pltpu.sync_copy(wi_hbm.at[pl.ds(base_b, ch)], ibw)

            @pl.when(t > 0)
            def _():
                drain_store(base_b - 2 * ch, bqb, ssb)

            g1 = pltpu.async_copy(qc_hbm.at[ibh], bqb, gsb)
            g2 = pltpu.async_copy(cc_hbm.at[ibw], bcb, gsb)
            drain_gathers(base_a, bqa, bca, gsa)
            vsum(bqa, bca)
            sa1 = pltpu.async_copy(bqa, qs_out.at[pl.ds(base_a, ch)], ssa)
            g1.wait()
            g2.wait()
            vsum(bqb, bcb)
            pltpu.async_copy(bqb, qs_out.at[pl.ds(base_b, ch)], ssb)
            pltpu.sync_copy(hi_hbm.at[pl.ds(base_a2, ch)], iah)
            pltpu.sync_copy(wi_hbm.at[pl.ds(base_a2, ch)], iaw)
            sa1.wait()
            pltpu.async_copy(qc_hbm.at[iah], bqa, gsa)
            pltpu.async_copy(cc_hbm.at[iaw], bca, gsa)
            return ()

        lax.fori_loop(0, npairs, pair, (), unroll=False)
        base_l = pl.multiple_of(wbase + (nchunks - 1) * ch, 8)
        drain_store(base_l - ch, bqb, ssb)
        drain_gathers(base_l, bqa, bca, gsa)
        vsum(bqa, bca)
        pltpu.sync_copy(bqa, qs_out.at[pl.ds(base_l, ch)])

    return gather


def _edge_body(dif_ref, qs_ref, w1p_ref, b1_ref, w2_ref,
               g2_ref, b2_ref, w1d_ref, g1_ref, b1c_ref, wc2_ref, c_ref):
    dp8 = jnp.dot(dif_ref[...], w1p_ref[...],
                  preferred_element_type=jnp.float32)
    dpre = jnp.concatenate([dp8[:, j * 128:(j + 1) * 128] for j in range(8)],
                           axis=0)
    d = jax.nn.relu(dpre + b1_ref[...])
    d = jax.nn.relu(_gn(jnp.dot(d, w2_ref[...],
                                preferred_element_type=jnp.float32),
                        g2_ref[...], b2_ref[...]))
    cpre = (jnp.dot(d, w1d_ref[...], preferred_element_type=jnp.float32)
            + qs_ref[...])
    c = jax.nn.relu(_gn(cpre, g1_ref[...], b1c_ref[...]))
    c_ref[...] = jnp.dot(c, wc2_ref[...], preferred_element_type=jnp.float32)


def _edge_mlp(dif, qs, w1p, b1, w2, g2, b2, w1d, g1, b1c, wc2, be):
    e, d = qs.shape
    grid = e // be
    row = pl.BlockSpec((be, d), lambda i: (i, 0))
    row16 = pl.BlockSpec((be // 8, 128), lambda i: (i, 0))
    mat = pl.BlockSpec((d, d), lambda i: (0, 0))
    mat16 = pl.BlockSpec((128, 8 * d), lambda i: (0, 0))
    vec = pl.BlockSpec((1, d), lambda i: (0, 0))
    return pl.pallas_call(
        _edge_body,
        grid=(grid,),
        in_specs=[row16, row, mat16, vec, mat, vec, vec, mat,
                  vec, vec, mat],
        out_specs=row,
        out_shape=jax.ShapeDtypeStruct((e, d), jnp.float32),
    )(dif, qs, w1p, b1.reshape(1, d), w2, g2.reshape(1, d),
      b2.reshape(1, d), w1d, g1.reshape(1, d), b1c.reshape(1, d), wc2)


def _make_scatter(n, e, d, nc, ns, ch, nsl):
    esl = e // nsl
    ew = esl // (nc * ns)
    nchunks = ew // ch
    rpt = 1000
    mesh = plsc.VectorSubcoreMesh(core_axis_name="c", subcore_axis_name="s")

    assert nchunks % 2 == 1, "pipelined schedule assumes an odd chunk count"
    npairs = (nchunks - 1) // 2

    @functools.partial(
        pl.kernel,
        mesh=mesh,
        out_type=jax.ShapeDtypeStruct((nc * n, d), jnp.float32),
        scratch_types=[
            pltpu.VMEM_SHARED((n, d), jnp.float32),
            pltpu.VMEM((ch,), jnp.int32),
            pltpu.VMEM((ch,), jnp.int32),
            pltpu.VMEM((ch, d), jnp.float32),
            pltpu.VMEM((ch, d), jnp.float32),
            pltpu.SemaphoreType.DMA,
            pltpu.SemaphoreType.DMA,
        ],
        compiler_params=pltpu.CompilerParams(needs_layout_passes=False),
    )
    def scatter(*refs):
        c_list = refs[:nsl]
        hi_hbm = refs[nsl]
        zeros_hbm = refs[nsl + 1]
        part_out = refs[nsl + 2]
        acc, ia, ib, bva, bvb, sla, slb = refs[nsl + 3:]
        cid = lax.axis_index("c")
        sid = lax.axis_index("s")
        wid = cid * ns + sid

        @pl.when(sid == 0)
        def _():
            pltpu.sync_copy(zeros_hbm, acc)

        plsc.subcore_barrier()

        for s in range(nsl):
            c_hbm = c_list[s]
            wbase = pl.multiple_of(wid * ew, 8)
            hbase = pl.multiple_of(s * esl + wid * ew, 8)

            def fire_loads(base, ix, bv, sem):
                pltpu.async_copy(hi_hbm.at[pl.ds(base + (hbase - wbase), ch)],
                                 ix, sem)
                pltpu.async_copy(c_hbm.at[pl.ds(base, ch)], bv, sem)

            def drain_loads(base, ix, bv, sem):
                pltpu.make_async_copy(
                    hi_hbm.at[pl.ds(base + (hbase - wbase), ch)],
                    ix, sem).wait()
                pltpu.make_async_copy(c_hbm.at[pl.ds(base, ch)], bv,
                                      sem).wait()

            fire_loads(wbase, ia, bva, sla)

            def pair(t, _):
                base_a = pl.multiple_of(wbase + 2 * t * ch, 8)
                base_b = pl.multiple_of(base_a + ch, 8)
                base_a2 = pl.multiple_of(base_b + ch, 8)
                fire_loads(base_b, ib, bvb, slb)
                drain_loads(base_a, ia, bva, sla)
                pltpu.sync_copy(bva, acc.at[ia], add=True)
                fire_loads(base_a2, ia, bva, sla)
                drain_loads(base_b, ib, bvb, slb)
                pltpu.sync_copy(bvb, acc.at[ib], add=True)
                return ()

            lax.fori_loop(0, npairs, pair, (), unroll=False)
            base_l = pl.multiple_of(wbase + (nchunks - 1) * ch, 8)
            drain_loads(base_l, ia, bva, sla)
            pltpu.sync_copy(bva, acc.at[ia], add=True)
        plsc.subcore_barrier()

        @pl.when(sid < n // rpt)
        def _():
            rbase = pl.multiple_of(sid * rpt, 8)
            pltpu.sync_copy(acc.at[pl.ds(rbase, rpt)],
                            part_out.at[pl.ds(cid * n + rbase, rpt)])

    return scatter


def _final_body(a0_ref, p0_ref, p1_ref, p2_ref, p3_ref, res_ref, ng_ref,
                nb_ref, lw_ref, lg_ref, lb_ref, out_ref):
    a = (a0_ref[...] + p0_ref[...] + p1_ref[...] + p2_ref[...]
         + p3_ref[...])
    a = jax.nn.relu(_gn(a, ng_ref[...], nb_ref[...]))
    a = _gn(jnp.dot(a, lw_ref[...], preferred_element_type=jnp.float32),
            lg_ref[...], lb_ref[...])
    out_ref[...] = jax.nn.relu(a + res_ref[...])


def _node_final(a0, p0, p1, p2, p3, res, norm_g, norm_b, lin_w, lin_g,
                lin_b, bn):
    n, d = a0.shape
    grid = n // bn
    row = pl.BlockSpec((bn, d), lambda i: (i, 0))
    mat = pl.BlockSpec((d, d), lambda i: (0, 0))
    vec = pl.BlockSpec((1, d), lambda i: (0, 0))
    return pl.pallas_call(
        _final_body,
        grid=(grid,),
        in_specs=[row, row, row, row, row, row, vec, vec, mat, vec, vec],
        out_specs=row,
        out_shape=jax.ShapeDtypeStruct((n, d), jnp.float32),
    )(a0, p0, p1, p2, p3, res, norm_g.reshape(1, d), norm_b.reshape(1, d),
      lin_w, lin_g.reshape(1, d), lin_b.reshape(1, d))


def kernel(agts, agt_ctrs, ctx, ctx_ctrs, edge_index, dist_w1, dist_b1,
           dist_w2, dist_g2, dist_b2, query_w, query_g, query_b, ctx_w1,
           ctx_g1, ctx_b1, ctx_w2, agt_w, norm_g, norm_b, lin_w, lin_g,
           lin_b):
    n, d = agts.shape
    e = edge_index.shape[1]
    ncc = agt_ctrs.shape[1]
    nc, ns = 2, 16

    hi = edge_index[0]
    wi = edge_index[1]
    w1d, w1q, w1c = ctx_w1[:d], ctx_w1[d:2 * d], ctx_w1[2 * d:]

    actr_f = agt_ctrs.reshape(-1)
    cctr_f = ctx_ctrs.reshape(-1)
    w1p16 = jnp.zeros((16, d), jnp.float32).at[:ncc].set(dist_w1)
    w1p = (jnp.eye(8, dtype=jnp.float32)[:, None, :, None]
           * w1p16[None, :, None, :]).reshape(128, 8 * d)
    zeros_nd = jnp.zeros((n, d), jnp.float32)

    qc, cc, a0 = _node_pre(agts, ctx, query_w, query_g, query_b, w1q, w1c,
                           agt_w, bn=2000)

    nsl = 5
    esl = e // nsl
    gather = _make_gather(n, esl, d, nc, ns, ch=80, be=3200)
    edge = functools.partial(_edge_mlp, be=3200)
    cs = []
    for s in range(nsl):
        qs, dif_p = gather(qc, cc, actr_f, cctr_f,
                           hi[s * esl:(s + 1) * esl],
                           wi[s * esl:(s + 1) * esl])
        cs.append(edge(dif_p, qs, w1p, dist_b1,
                       dist_w2, dist_g2, dist_b2, w1d, ctx_g1, ctx_b1,
                       ctx_w2))
    scatter_a = _make_scatter(n, 3 * esl, d, nc, ns, ch=80, nsl=3)
    scatter_b = _make_scatter(n, 2 * esl, d, nc, ns, ch=80, nsl=2)
    parts_a = scatter_a(*cs[:3], hi[:3 * esl], zeros_nd)
    parts_b = scatter_b(*cs[3:], hi[3 * esl:], zeros_nd)

    return _node_final(a0, parts_a[:n], parts_a[n:], parts_b[:n],
                       parts_b[n:], agts, norm_g, norm_b, lin_w, lin_g,
                       lin_b, bn=2000)


# speedup vs baseline: 1.1740x; 1.0877x over previous
import functools

import jax
import jax.numpy as jnp
from jax import lax
from jax.experimental import pallas as pl
from jax.experimental.pallas import tpu as pltpu
from jax.experimental.pallas import tpu_sc as plsc


def _gn(x, g, b, eps=1e-5):
    mu = jnp.mean(x, axis=-1, keepdims=True)
    var = jnp.mean((x - mu) ** 2, axis=-1, keepdims=True)
    return (x - mu) / jnp.sqrt(var + eps) * g + b


def _pre_body(agts_ref, ctx_ref, qw_ref, qg_ref, qb_ref, w1q_ref, w1c_ref,
              aw_ref, qc_ref, cc_ref, a0_ref):
    x = agts_ref[...]
    q = jax.nn.relu(_gn(jnp.dot(x, qw_ref[...],
                                preferred_element_type=jnp.float32),
                        qg_ref[...], qb_ref[...]))
    qc_ref[...] = jnp.dot(q, w1q_ref[...], preferred_element_type=jnp.float32)
    cc_ref[...] = jnp.dot(ctx_ref[...], w1c_ref[...],
                          preferred_element_type=jnp.float32)
    a0_ref[...] = jnp.dot(x, aw_ref[...], preferred_element_type=jnp.float32)


def _node_pre(agts, ctx, query_w, query_g, query_b, w1q, w1c, agt_w, bn):
    n, d = agts.shape
    grid = n // bn
    row = pl.BlockSpec((bn, d), lambda i: (i, 0))
    mat = pl.BlockSpec((d, d), lambda i: (0, 0))
    vec = pl.BlockSpec((1, d), lambda i: (0, 0))
    out = jax.ShapeDtypeStruct((n, d), jnp.float32)
    outh = jax.ShapeDtypeStruct((n, d), jnp.bfloat16)
    return pl.pallas_call(
        _pre_body,
        grid=(grid,),
        in_specs=[row, row, mat, vec, vec, mat, mat, mat],
        out_specs=[row, row, row],
        out_shape=[out, out, out],
    )(agts, ctx, query_w, query_g.reshape(1, d), query_b.reshape(1, d),
      w1q, w1c, agt_w)


def _make_gather(n, e, d, nc, ns, ch, be):
    ew = e // (nc * ns)
    nchunks = ew // ch
    assert nchunks % 2 == 1, "pipelined schedule assumes an odd chunk count"
    npairs = (nchunks - 1) // 2
    nblk = e // be
    rpb = be // 8
    rchunk = 80
    mesh = plsc.VectorSubcoreMesh(core_axis_name="c", subcore_axis_name="s")

    @functools.partial(
        pl.kernel,
        mesh=mesh,
        out_type=[
            jax.ShapeDtypeStruct((e, d), jnp.float32),
            jax.ShapeDtypeStruct((e // 8, 128), jnp.float32),
        ],
        scratch_types=[
            pltpu.VMEM((ew,), jnp.int32),
            pltpu.VMEM((ew,), jnp.int32),
            pltpu.VMEM((ch, d), jnp.float32),
            pltpu.VMEM((ch, d), jnp.float32),
            pltpu.VMEM((ch, d), jnp.float32),
            pltpu.VMEM((ch, d), jnp.float32),
            pltpu.VMEM((rchunk, 128), jnp.float32),
            pltpu.VMEM((be,), jnp.int32),
            pltpu.VMEM((be,), jnp.int32),
            pltpu.VMEM((2 * n,), jnp.float32),
            pltpu.VMEM((2 * n,), jnp.float32),
            pltpu.SemaphoreType.DMA,
            pltpu.SemaphoreType.DMA,
            pltpu.SemaphoreType.DMA,
            pltpu.SemaphoreType.DMA,
        ],
        compiler_params=pltpu.CompilerParams(needs_layout_passes=False),
    )
    def gather(qc_hbm, cc_hbm, actr_hbm, cctr_hbm, hi_hbm, wi_hbm,
               qs_out, dif_out,
               ihb, iwb, bqa, bca, bqb, bcb, dbuf, dih, diw,
               actr_v, cctr_v, gsa, gsb, ssa, ssb):
        wid = lax.axis_index("c") * ns + lax.axis_index("s")
        wbase = pl.multiple_of(wid * ew, 8)
        iota = lax.iota(jnp.int32, 16)

        pltpu.sync_copy(hi_hbm.at[pl.ds(wbase, ew)], ihb)
        pltpu.sync_copy(wi_hbm.at[pl.ds(wbase, ew)], iwb)
        pltpu.async_copy(qc_hbm.at[ihb.at[pl.ds(0, ch)]], bqa, gsa)
        pltpu.async_copy(cc_hbm.at[iwb.at[pl.ds(0, ch)]], bca, gsa)
        pltpu.sync_copy(actr_hbm, actr_v)
        pltpu.sync_copy(cctr_hbm, cctr_v)

        def zero_row(i, _):
            for k in range(8):
                dbuf[i, pl.ds(k * 16, 16)] = jnp.zeros((16,), jnp.float32)
            return ()

        lax.fori_loop(0, rchunk, zero_row, (), unroll=False)

        @pl.when(wid < nblk)
        def _():
            dbase = pl.multiple_of(wid * be, 8)
            pltpu.sync_copy(hi_hbm.at[pl.ds(dbase, be)], dih)
            pltpu.sync_copy(wi_hbm.at[pl.ds(dbase, be)], diw)
            for rc in range(rpb // rchunk):
                for j in range(8):
                    for g in range(rchunk // 16):
                        off = j * rpb + rc * rchunk + g * 16
                        rh = dih[pl.ds(off, 16)] * 2
                        rw = diw[pl.ds(off, 16)] * 2
                        d0 = (plsc.load_gather(actr_v, [rh])
                              - plsc.load_gather(cctr_v, [rw]))
                        d1 = (plsc.load_gather(actr_v, [rh + 1])
                              - plsc.load_gather(cctr_v, [rw + 1]))
                        rows = g * 16 + iota
                        cols = jnp.full((16,), 16 * j, jnp.int32)
                        plsc.store_scatter(dbuf, [rows, cols], d0)
                        plsc.store_scatter(dbuf, [rows, cols + 1], d1)
                rowb = pl.multiple_of(wid * rpb + rc * rchunk, 8)
                pltpu.sync_copy(dbuf, dif_out.at[pl.ds(rowb, rchunk)])

        def vsum(bq, bc):
            def row(r, _):
                for k in range(8):
                    sl = pl.ds(k * 16, 16)
                    bq[r, sl] = bq[r, sl] + bc[r, sl]
                return ()

            lax.fori_loop(0, ch, row, (), unroll=False)

        def drain_store(base, bq, sem):
            pltpu.make_async_copy(bq, qs_out.at[pl.ds(base, ch)], sem).wait()

        def drain_gathers(base, bq, bc, sem):
            pltpu.make_async_copy(qs_out.at[pl.ds(base, ch)], bq, sem).wait()
            pltpu.make_async_copy(qs_out.at[pl.ds(base, ch)], bc, sem).wait()

        def pair(t, _):
            base_a = pl.multiple_of(wbase + 2 * t * ch, 8)
            base_b = pl.multiple_of(base_a + ch, 8)
            off_b = pl.multiple_of(2 * t * ch + ch, 8)
            off_a2 = pl.multiple_of(off_b + ch, 8)
            @pl.when(t > 0)
            def _():
                drain_store(base_b - 2 * ch, bqb, ssb)

            g1 = pltpu.async_copy(qc_hbm.at[ihb.at[pl.ds(off_b, ch)]],
                                  bqb, gsb)
            g2 = pltpu.async_copy(cc_hbm.at[iwb.at[pl.ds(off_b, ch)]],
                                  bcb, gsb)
            drain_gathers(base_a, bqa, bca, gsa)
            vsum(bqa, bca)
            sa1 = pltpu.async_copy(bqa, qs_out.at[pl.ds(base_a, ch)], ssa)
            g1.wait()
            g2.wait()
            vsum(bqb, bcb)
            pltpu.async_copy(bqb, qs_out.at[pl.ds(base_b, ch)], ssb)
            sa1.wait()
            pltpu.async_copy(qc_hbm.at[ihb.at[pl.ds(off_a2, ch)]], bqa, gsa)
            pltpu.async_copy(cc_hbm.at[iwb.at[pl.ds(off_a2, ch)]], bca, gsa)
            return ()

        lax.fori_loop(0, npairs, pair, (), unroll=False)
        base_l = pl.multiple_of(wbase + (nchunks - 1) * ch, 8)
        drain_store(base_l - ch, bqb, ssb)
        drain_gathers(base_l, bqa, bca, gsa)
        vsum(bqa, bca)
        pltpu.sync_copy(bqa, qs_out.at[pl.ds(base_l, ch)])

    return gather


def _edge_body(dif_ref, qs_ref, w1p_ref, b1_ref, w2_ref,
               g2_ref, b2_ref, w1d_ref, g1_ref, b1c_ref, wc2_ref, c_ref):
    dp8 = jnp.dot(dif_ref[...], w1p_ref[...],
                  preferred_element_type=jnp.float32)
    dpre = jnp.concatenate([dp8[:, j * 128:(j + 1) * 128] for j in range(8)],
                           axis=0)
    d = jax.nn.relu(dpre + b1_ref[...])
    d = jax.nn.relu(_gn(jnp.dot(d, w2_ref[...],
                                preferred_element_type=jnp.float32),
                        g2_ref[...], b2_ref[...]))
    cpre = (jnp.dot(d, w1d_ref[...], preferred_element_type=jnp.float32)
            + qs_ref[...])
    c = jax.nn.relu(_gn(cpre, g1_ref[...], b1c_ref[...]))
    c_ref[...] = jnp.dot(c, wc2_ref[...], preferred_element_type=jnp.float32)


def _edge_mlp(dif, qs, w1p, b1, w2, g2, b2, w1d, g1, b1c, wc2, be):
    e, d = qs.shape
    grid = e // be
    row = pl.BlockSpec((be, d), lambda i: (i, 0))
    row16 = pl.BlockSpec((be // 8, 128), lambda i: (i, 0))
    mat = pl.BlockSpec((d, d), lambda i: (0, 0))
    mat16 = pl.BlockSpec((128, 8 * d), lambda i: (0, 0))
    vec = pl.BlockSpec((1, d), lambda i: (0, 0))
    return pl.pallas_call(
        _edge_body,
        grid=(grid,),
        in_specs=[row16, row, mat16, vec, mat, vec, vec, mat,
                  vec, vec, mat],
        out_specs=row,
        out_shape=jax.ShapeDtypeStruct((e, d), jnp.float32),
    )(dif, qs, w1p, b1.reshape(1, d), w2, g2.reshape(1, d),
      b2.reshape(1, d), w1d, g1.reshape(1, d), b1c.reshape(1, d), wc2)


def _make_scatter(n, e, d, nc, ns, ch, nsl):
    esl = e // nsl
    ew = esl // (nc * ns)
    nchunks = ew // ch
    rpt = 1000
    mesh = plsc.VectorSubcoreMesh(core_axis_name="c", subcore_axis_name="s")

    assert nchunks % 2 == 1, "pipelined schedule assumes an odd chunk count"
    npairs = (nchunks - 1) // 2

    @functools.partial(
        pl.kernel,
        mesh=mesh,
        out_type=jax.ShapeDtypeStruct((nc * n, d), jnp.float32),
        scratch_types=[
            pltpu.VMEM_SHARED((n, d), jnp.float32),
            pltpu.VMEM((ch,), jnp.int32),
            pltpu.VMEM((ch,), jnp.int32),
            pltpu.VMEM((ch, d), jnp.float32),
            pltpu.VMEM((ch, d), jnp.float32),
            pltpu.SemaphoreType.DMA,
            pltpu.SemaphoreType.DMA,
        ],
        compiler_params=pltpu.CompilerParams(needs_layout_passes=False),
    )
    def scatter(*refs):
        c_list = refs[:nsl]
        hi_hbm = refs[nsl]
        zeros_hbm = refs[nsl + 1]
        part_out = refs[nsl + 2]
        acc, ia, ib, bva, bvb, sla, slb = refs[nsl + 3:]
        cid = lax.axis_index("c")
        sid = lax.axis_index("s")
        wid = cid * ns + sid

        @pl.when(sid == 0)
        def _():
            pltpu.sync_copy(zeros_hbm, acc)

        plsc.subcore_barrier()

        for s in range(nsl):
            c_hbm = c_list[s]
            wbase = pl.multiple_of(wid * ew, 8)
            hbase = pl.multiple_of(s * esl + wid * ew, 8)

            def fire_loads(base, ix, bv, sem):
                pltpu.async_copy(hi_hbm.at[pl.ds(base + (hbase - wbase), ch)],
                                 ix, sem)
                pltpu.async_copy(c_hbm.at[pl.ds(base, ch)], bv, sem)

            def drain_loads(base, ix, bv, sem):
                pltpu.make_async_copy(
                    hi_hbm.at[pl.ds(base + (hbase - wbase), ch)],
                    ix, sem).wait()
                pltpu.make_async_copy(c_hbm.at[pl.ds(base, ch)], bv,
                                      sem).wait()

            fire_loads(wbase, ia, bva, sla)

            def pair(t, _):
                base_a = pl.multiple_of(wbase + 2 * t * ch, 8)
                base_b = pl.multiple_of(base_a + ch, 8)
                base_a2 = pl.multiple_of(base_b + ch, 8)
                fire_loads(base_b, ib, bvb, slb)
                drain_loads(base_a, ia, bva, sla)
                pltpu.sync_copy(bva, acc.at[ia], add=True)
                fire_loads(base_a2, ia, bva, sla)
                drain_loads(base_b, ib, bvb, slb)
                pltpu.sync_copy(bvb, acc.at[ib], add=True)
                return ()

            lax.fori_loop(0, npairs, pair, (), unroll=False)
            base_l = pl.multiple_of(wbase + (nchunks - 1) * ch, 8)
            drain_loads(base_l, ia, bva, sla)
            pltpu.sync_copy(bva, acc.at[ia], add=True)
        plsc.subcore_barrier()

        @pl.when(sid < n // rpt)
        def _():
            rbase = pl.multiple_of(sid * rpt, 8)
            pltpu.sync_copy(acc.at[pl.ds(rbase, rpt)],
                            part_out.at[pl.ds(cid * n + rbase, rpt)])

    return scatter


def _final_body(a0_ref, p0_ref, p1_ref, p2_ref, p3_ref, res_ref, ng_ref,
                nb_ref, lw_ref, lg_ref, lb_ref, out_ref):
    a = (a0_ref[...] + p0_ref[...] + p1_ref[...] + p2_ref[...]
         + p3_ref[...])
    a = jax.nn.relu(_gn(a, ng_ref[...], nb_ref[...]))
    a = _gn(jnp.dot(a, lw_ref[...], preferred_element_type=jnp.float32),
            lg_ref[...], lb_ref[...])
    out_ref[...] = jax.nn.relu(a + res_ref[...])


def _node_final(a0, p0, p1, p2, p3, res, norm_g, norm_b, lin_w, lin_g,
                lin_b, bn):
    n, d = a0.shape
    grid = n // bn
    row = pl.BlockSpec((bn, d), lambda i: (i, 0))
    mat = pl.BlockSpec((d, d), lambda i: (0, 0))
    vec = pl.BlockSpec((1, d), lambda i: (0, 0))
    return pl.pallas_call(
        _final_body,
        grid=(grid,),
        in_specs=[row, row, row, row, row, row, vec, vec, mat, vec, vec],
        out_specs=row,
        out_shape=jax.ShapeDtypeStruct((n, d), jnp.float32),
    )(a0, p0, p1, p2, p3, res, norm_g.reshape(1, d), norm_b.reshape(1, d),
      lin_w, lin_g.reshape(1, d), lin_b.reshape(1, d))


def kernel(agts, agt_ctrs, ctx, ctx_ctrs, edge_index, dist_w1, dist_b1,
           dist_w2, dist_g2, dist_b2, query_w, query_g, query_b, ctx_w1,
           ctx_g1, ctx_b1, ctx_w2, agt_w, norm_g, norm_b, lin_w, lin_g,
           lin_b):
    n, d = agts.shape
    e = edge_index.shape[1]
    ncc = agt_ctrs.shape[1]
    nc, ns = 2, 16

    hi = edge_index[0]
    wi = edge_index[1]
    w1d, w1q, w1c = ctx_w1[:d], ctx_w1[d:2 * d], ctx_w1[2 * d:]

    actr_f = agt_ctrs.reshape(-1)
    cctr_f = ctx_ctrs.reshape(-1)
    w1p16 = jnp.zeros((16, d), jnp.float32).at[:ncc].set(dist_w1)
    w1p = (jnp.eye(8, dtype=jnp.float32)[:, None, :, None]
           * w1p16[None, :, None, :]).reshape(128, 8 * d)
    zeros_nd = jnp.zeros((n, d), jnp.float32)

    qc, cc, a0 = _node_pre(agts, ctx, query_w, query_g, query_b, w1q, w1c,
                           agt_w, bn=2000)

    nsl = 5
    esl = e // nsl
    gather = _make_gather(n, esl, d, nc, ns, ch=80, be=3200)
    edge = functools.partial(_edge_mlp, be=3200)
    cs = []
    for s in range(nsl):
        qs, dif_p = gather(qc, cc, actr_f, cctr_f,
                           hi[s * esl:(s + 1) * esl],
                           wi[s * esl:(s + 1) * esl])
        cs.append(edge(dif_p, qs, w1p, dist_b1,
                       dist_w2, dist_g2, dist_b2, w1d, ctx_g1, ctx_b1,
                       ctx_w2))
    scatter_a = _make_scatter(n, 3 * esl, d, nc, ns, ch=80, nsl=3)
    scatter_b = _make_scatter(n, 2 * esl, d, nc, ns, ch=80, nsl=2)
    parts_a = scatter_a(*cs[:3], hi[:3 * esl], zeros_nd)
    parts_b = scatter_b(*cs[3:], hi[3 * esl:], zeros_nd)

    return _node_final(a0, parts_a[:n], parts_a[n:], parts_b[:n],
                       parts_b[n:], agts, norm_g, norm_b, lin_w, lin_g,
                       lin_b, bn=2000)
